# baseline (speedup 1.0000x reference)
import jax
import jax.numpy as jnp
from jax import lax
from jax.experimental import pallas as pl
from jax.experimental.pallas import tpu as pltpu
from jax.experimental.pallas import tpu_sc as plsc

B, L, D = 16, 2048, 1024
LANES = 16
HALF = D // 2
NVEC = HALF // LANES
CH = 64
SEQ_PER_SC = B // 2
WPH = 8


def _body(xs_hbm, mi_hbm, mf_hbm, out_hbm, part_hbm, lbuf, fbuf, buf0, buf1,
          obuf, sem0, sem1, semw):
    c = lax.axis_index("c")
    s = lax.axis_index("s")
    h = s % 2
    wk = s // 2
    col0 = h * HALF

    pltpu.sync_copy(mi_hbm.at[c, s], lbuf)
    pltpu.sync_copy(mf_hbm.at[c, s], fbuf)
    spans = lbuf[...]
    inv = fbuf[...][0]

    def issue(t0, buf, sem):
        t0c = jnp.minimum(t0, L - CH)
        pltpu.make_async_copy(
            xs_hbm.at[b, pl.ds(t0c, CH), pl.ds(col0, HALF)], buf, sem
        ).start()

    def wait(buf, sem):
        pltpu.make_async_copy(
            xs_hbm.at[b, pl.ds(0, CH), pl.ds(col0, HALF)], buf, sem
        ).wait()

    for j in range(SEQ_PER_SC):
        b = c * SEQ_PER_SC + j
        t_lo = spans[j]
        t_len = spans[8 + j]
        t_end = t_lo + t_len
        al_lo = (t_lo // 8) * 8
        nch = jnp.where(t_len > 0, (t_end - al_lo + CH - 1) // CH, 0)

        def accum(i, buf, accs):
            t0 = al_lo + i * CH
            t0c = jnp.minimum(t0, L - CH)
            start = jnp.maximum(t_lo, t0)
            off = start - t0c
            nv = jnp.clip(jnp.minimum(t_end, t0 + CH) - start, 0, CH)

            def row_body(r, a):
                row = off + r
                return tuple(
                    a[v] + buf[row, pl.ds(v * LANES, LANES)]
                    for v in range(NVEC)
                )

            return lax.fori_loop(0, nv, row_body, accs)

        @pl.when(0 < nch)
        def _():
            issue(al_lo, buf0, sem0)

        @pl.when(1 < nch)
        def _():
            issue(al_lo + CH, buf1, sem1)

        def pair_body(i2, accs):
            ca = 2 * i2
            wait(buf0, sem0)
            accs = accum(ca, buf0, accs)

            @pl.when(ca + 2 < nch)
            def _():
                issue(al_lo + (ca + 2) * CH, buf0, sem0)

            @pl.when(ca + 1 < nch)
            def _():
                wait(buf1, sem1)

            accs = accum(ca + 1, buf1, accs)

            @pl.when(ca + 3 < nch)
            def _():
                issue(al_lo + (ca + 3) * CH, buf1, sem1)

            return accs

        accs = tuple(jnp.zeros((LANES,), jnp.float32) for _ in range(NVEC))
        accs = lax.fori_loop(0, (nch + 1) // 2, pair_body, accs)
        for v in range(NVEC):
            obuf[j, pl.ds(v * LANES, LANES)] = accs[v]
        pltpu.make_async_copy(obuf.at[j], part_hbm.at[b, h, wk], semw).start()

    for j in range(SEQ_PER_SC):
        pltpu.make_async_copy(
            obuf.at[j], part_hbm.at[c * SEQ_PER_SC + j, h, wk], semw
        ).wait()

    plsc.subcore_barrier()

    b_f = c * SEQ_PER_SC + wk
    pltpu.sync_copy(part_hbm.at[b_f, h], buf0.at[pl.ds(0, WPH)])
    accs = tuple(jnp.zeros((LANES,), jnp.float32) for _ in range(NVEC))
    for r in range(WPH):
        accs = tuple(
            accs[v] + buf0[r, pl.ds(v * LANES, LANES)] for v in range(NVEC)
        )
    for v in range(NVEC):
        obuf[0, pl.ds(v * LANES, LANES)] = accs[v] * inv
    pltpu.sync_copy(obuf.at[0], out_hbm.at[b_f, pl.ds(col0, HALF)])


@jax.jit
def _mean_pool(xs, mi, mf):
    kern = pl.kernel(
        _body,
        out_type=(
            jax.ShapeDtypeStruct((B, D), jnp.float32),
            jax.ShapeDtypeStruct((B, 2, WPH, HALF), jnp.float32),
        ),
        mesh=plsc.VectorSubcoreMesh(core_axis_name="c", subcore_axis_name="s"),
        scratch_types=[
            pltpu.VMEM((LANES,), jnp.int32),
            pltpu.VMEM((LANES,), jnp.float32),
            pltpu.VMEM((CH, HALF), jnp.float32),
            pltpu.VMEM((CH, HALF), jnp.float32),
            pltpu.VMEM((SEQ_PER_SC, HALF), jnp.float32),
            pltpu.SemaphoreType.DMA,
            pltpu.SemaphoreType.DMA,
            pltpu.SemaphoreType.DMA,
        ],
    )
    out, _ = kern(xs, mi, mf)
    return out


def kernel(xs, xs_len):
    lens = xs_len.astype(jnp.int32)
    inv = 1.0 / lens.astype(jnp.float32)
    ls = lens.reshape(2, SEQ_PER_SC)
    cum = jnp.concatenate(
        [jnp.zeros((2, 1), jnp.int32), jnp.cumsum(ls, axis=1)], axis=1
    )
    tot = cum[:, SEQ_PER_SC]
    rr = jnp.arange(WPH)
    lo = rr[None, :] * tot[:, None] // WPH
    hi = (rr[None, :] + 1) * tot[:, None] // WPH
    t_lo = jnp.clip(lo[:, :, None] - cum[:, None, :SEQ_PER_SC],
                    0, ls[:, None, :])
    t_hi = jnp.clip(hi[:, :, None] - cum[:, None, :SEQ_PER_SC],
                    0, ls[:, None, :])
    ridx = jnp.arange(16) // 2
    mi = jnp.concatenate(
        [t_lo[:, ridx, :], (t_hi - t_lo)[:, ridx, :]], axis=-1
    ).astype(jnp.int32)
    cc = jnp.arange(2)[:, None]
    bmap = cc * SEQ_PER_SC + jnp.arange(16)[None, :] // 2
    mf = jnp.broadcast_to(inv[bmap][:, :, None], (2, 16, LANES))
    return _mean_pool(xs, mi, mf)
